# baseline (speedup 1.0000x reference)
import dataclasses
import functools

import jax
import jax.numpy as jnp
from jax import lax
from jax.experimental import pallas as pl
from jax.experimental.pallas import tpu as pltpu
from jax.experimental.pallas import tpu_sc as plsc

_LEAK = 0.9

_NC = 2
_NS = 16
_NW = _NC * _NS
_L = 16

def _sc_params():
    cp = pltpu.CompilerParams()
    fields = pltpu.CompilerParams.__dataclass_fields__
    if "use_tc_tiling_on_sc" in fields:
        cp = dataclasses.replace(cp, use_tc_tiling_on_sc=False)
    if "needs_layout_passes" in fields:
        cp = dataclasses.replace(cp, needs_layout_passes=False)
    return cp


_M = 80
_K = 8
_G = _M * _K

_PM = 128
_PK = 8
_PG = _PM * _PK

_SK = 8
_SG = _SK * _PM
_CAPB = 11264
_MAXB = _CAPB // _SG


def _lane_bcast(vec, lane):
    idx = jnp.full((_L,), lane, dtype=jnp.int32)
    dnums = lax.GatherDimensionNumbers(
        offset_dims=(), collapsed_slice_dims=(0,), start_index_map=(0,))
    return lax.gather(vec, idx[:, None], dnums, slice_sizes=(1,),
                      mode=lax.GatherScatterMode.PROMISE_IN_BOUNDS)


def _degrees(row2d, ew2d, ones, npad):
    nrows = row2d.shape[0]
    nb = nrows // (_PK * _NW)
    dstripe = npad // _NS
    mesh = plsc.VectorSubcoreMesh(core_axis_name="c", subcore_axis_name="s")

    @functools.partial(
        pl.kernel,
        out_type=jax.ShapeDtypeStruct((_NC, npad), jnp.float32),
        mesh=mesh,
        compiler_params=_sc_params(),
        scratch_types=[
            pltpu.VMEM((nb, _PK, _PM), jnp.int32),
            pltpu.VMEM((nb, _PK, _PM), jnp.float32),
            pltpu.SemaphoreType.DMA,
            pltpu.SemaphoreType.DMA,
            pltpu.VMEM_SHARED((npad,), jnp.float32),
        ],
    )
    def kern(row_hbm, ew_hbm, ones_hbm, out_hbm, row_v, ew_v, semi, sems,
             deg_sh):
        c = lax.axis_index("c")
        s = lax.axis_index("s")
        wid = c * _NS + s
        doff = pl.multiple_of(s * dstripe, 8)
        loads = []
        for b in range(nb):
            r0 = pl.multiple_of((wid + b * _NW) * _PK, 8)
            loads.append(pltpu.async_copy(row_hbm.at[pl.ds(r0, _PK)],
                                          row_v.at[b], semi))
            loads.append(pltpu.async_copy(ew_hbm.at[pl.ds(r0, _PK)],
                                          ew_v.at[b], semi))
        pltpu.sync_copy(ones_hbm.at[pl.ds(doff, dstripe)],
                        deg_sh.at[pl.ds(s * dstripe, dstripe)])
        plsc.subcore_barrier()
        for hh in loads:
            hh.wait()
        scats = []
        for b in range(nb):
            for j in range(_PK):
                scats.append(pltpu.async_copy(
                    ew_v.at[b].at[j], deg_sh.at[row_v.at[b].at[j]], sems,
                    add=True))
        for hh in scats:
            hh.wait()

        plsc.subcore_barrier()
        pltpu.sync_copy(deg_sh.at[pl.ds(s * dstripe, dstripe)],
                        out_hbm.at[c].at[pl.ds(doff, dstripe)])

    return kern(row2d, ew2d, ones)


def _bucket_edges(col, row2d, ew2d, npad):
    nrows = row2d.shape[0]
    nb = nrows // (_PK * _NW)
    half = npad // 2
    mesh = plsc.VectorSubcoreMesh(core_axis_name="c", subcore_axis_name="s")
    n_grp = nb * _PG // _L

    @functools.partial(
        pl.kernel,
        out_type=[
            jax.ShapeDtypeStruct((2, _NW, _CAPB), jnp.int32),
            jax.ShapeDtypeStruct((2, _NW, _MAXB, _SK, _PM), jnp.int32),
            jax.ShapeDtypeStruct((2, _NW, _CAPB), jnp.float32),
            jax.ShapeDtypeStruct((2, _NW, _L), jnp.int32),
        ],
        mesh=mesh,
        compiler_params=_sc_params(),
        scratch_types=[
            pltpu.VMEM((nb * _PK, _PM), jnp.int32),
            pltpu.VMEM((nb * _PK, _PM), jnp.int32),
            pltpu.VMEM((nb * _PK, _PM), jnp.float32),
            pltpu.VMEM((_CAPB,), jnp.int32),
            pltpu.VMEM((_CAPB // _PM, _PM), jnp.int32),
            pltpu.VMEM((_CAPB,), jnp.float32),
            pltpu.VMEM((_L,), jnp.int32),
            pltpu.SemaphoreType.DMA,
        ],
    )
    def kern(col_hbm, row_hbm, ew_hbm, qcol, qrow, qew, qcnt,
             ci_v, ri_v, wi_v, qc_v, qr_v, qw_v, cnt_v, semi):
        c = lax.axis_index("c")
        s = lax.axis_index("s")
        wid = c * _NS + s
        loads = []
        for b in range(nb):
            r0 = pl.multiple_of((wid + b * _NW) * _PK, 8)
            dst = pl.ds(b * _PK, _PK)
            loads.append(pltpu.async_copy(col_hbm.at[pl.ds(r0, _PK)],
                                          ci_v.at[dst], semi))
            loads.append(pltpu.async_copy(row_hbm.at[pl.ds(r0, _PK)],
                                          ri_v.at[dst], semi))
            loads.append(pltpu.async_copy(ew_hbm.at[pl.ds(r0, _PK)],
                                          wi_v.at[dst], semi))
        for hh in loads:
            hh.wait()

        lanes = lax.iota(jnp.int32, _L)
        for b in range(2):
            base = b * half

            def grp(g, cur):
                r16 = ri_v[g >> 3, pl.ds((g & 7) * _L, _L)]
                c16 = ci_v[g >> 3, pl.ds((g & 7) * _L, _L)]
                w16 = wi_v[g >> 3, pl.ds((g & 7) * _L, _L)]
                m = (r16 >= half) if b else (r16 < half)
                incl = plsc.cumsum(jnp.where(m, 1, 0).astype(jnp.int32))
                slot = incl - 1 + jnp.full((_L,), cur, jnp.int32)
                plsc.store_scatter(qc_v, [slot], c16, mask=m)
                plsc.store_scatter(
                    qr_v,
                    [lax.shift_right_logical(slot, 7),
                     lax.bitwise_and(slot, jnp.full((_L,), 127, jnp.int32))],
                    r16 - base, mask=m)
                plsc.store_scatter(qw_v, [slot], w16, mask=m)
                return cur + lax.reduce_max(incl, (0,))

            cur = lax.fori_loop(0, n_grp, grp, jnp.int32(0))

            npads = (_SG - cur % _SG) % _SG
            fillv = lax.bitwise_and(lanes + 1, jnp.full((_L,), 2047,
                                                        jnp.int32))

            @pl.loop(0, (npads + _L - 1) // _L)
            def _fill(kf):
                slot = jnp.full((_L,), cur + kf * _L, jnp.int32) + lanes
                plsc.store_scatter(qc_v, [slot], fillv)
                plsc.store_scatter(
                    qr_v,
                    [lax.shift_right_logical(slot, 7),
                     lax.bitwise_and(slot, jnp.full((_L,), 127, jnp.int32))],
                    fillv)
                plsc.store_scatter(qw_v, [slot],
                                   jnp.zeros((_L,), jnp.float32))

            nblk = (cur + npads) // _SG
            cnt_v[...] = jnp.full((_L,), nblk, jnp.int32)
            pltpu.sync_copy(cnt_v, qcnt.at[b].at[wid])

            for k in range(_MAXB):
                @pl.when(k < nblk)
                def _out(k=k):
                    e0 = k * _SG
                    pltpu.sync_copy(qc_v.at[pl.ds(e0, _SG)],
                                    qcol.at[b].at[wid].at[pl.ds(e0, _SG)])
                    pltpu.sync_copy(qw_v.at[pl.ds(e0, _SG)],
                                    qew.at[b].at[wid].at[pl.ds(e0, _SG)])
                    pltpu.sync_copy(qr_v.at[pl.ds(k * _SK, _SK)],
                                    qrow.at[b].at[wid].at[k])

    return kern(col.reshape(row2d.shape), row2d, ew2d)


@functools.lru_cache(maxsize=None)
def _make_step(npad, hdim):
    half = npad // 2
    own = half // _NS
    mesh = plsc.VectorSubcoreMesh(core_axis_name="c", subcore_axis_name="s")

    @functools.partial(
        pl.kernel,
        out_type=jax.ShapeDtypeStruct((npad, hdim), jnp.float32),
        mesh=mesh,
        compiler_params=_sc_params(),
        scratch_types=(
            [pltpu.VMEM((_SG,), jnp.int32) for _ in range(2)]
            + [pltpu.VMEM((_SK, _PM), jnp.int32) for _ in range(4)]
            + [pltpu.VMEM((_SG,), jnp.float32) for _ in range(2)]
            + [pltpu.VMEM((_SG, hdim), jnp.float32) for _ in range(2)]
            + [pltpu.SemaphoreType.DMA for _ in range(7)]
            + [pltpu.VMEM((_L,), jnp.int32),
               pltpu.VMEM((_L,), jnp.int32),
               pltpu.VMEM((own, hdim), jnp.float32),
               pltpu.VMEM((own, hdim), jnp.float32),
               pltpu.VMEM((own,), jnp.float32),
               pltpu.VMEM((hdim * hdim, _L), jnp.float32),
               pltpu.VMEM_SHARED((half, hdim), jnp.float32)]
        ),
    )
    def kern(qcol_hbm, qrow_hbm, qew_hbm, qcnt_hbm, h_hbm, xp_hbm, inv_hbm,
             wx_hbm, out_hbm, *scr):
        cols = scr[0:2]
        rowi = scr[2:6]
        ews = scr[6:8]
        rowsd = scr[8:10]
        semi = scr[10:12]
        semg = scr[12:14]
        sems = scr[14:16]
        semu = scr[16]
        cnt0, cnt1, hb, xpb, invb, wxb, agg_sh = scr[17:]
        acc = rowsd[0]
        c = lax.axis_index("c")
        s = lax.axis_index("s")
        nbase = c * half + s * own
        noff = pl.multiple_of(nbase, 8)

        up_h = [
            pltpu.async_copy(h_hbm.at[pl.ds(noff, own)], hb, semu),
            pltpu.async_copy(xp_hbm.at[pl.ds(noff, own)], xpb, semu),
            pltpu.async_copy(inv_hbm.at[pl.ds(noff, own)], invb, semu),
            pltpu.async_copy(wx_hbm, wxb, semu),
        ]
        pltpu.sync_copy(qcnt_hbm.at[c].at[2 * s], cnt0)
        pltpu.sync_copy(qcnt_hbm.at[c].at[2 * s + 1], cnt1)
        nblk0 = lax.reduce_max(cnt0[...], (0,))
        nblk1 = lax.reduce_max(cnt1[...], (0,))
        pltpu.sync_copy(h_hbm.at[pl.ds(noff, own)],
                        agg_sh.at[pl.ds(s * own, own)])
        plsc.subcore_barrier()

        seq = [(0, k) for k in range(_MAXB)] + [(1, k) for k in range(_MAXB)]
        nv = len(seq)

        def active(vi):
            r, k = seq[vi]
            return (k < nblk0) if r == 0 else (k < nblk1)

        def region(vi):
            return 2 * s + seq[vi][0]

        def issue_idx(vi):
            r, k = seq[vi]
            p = vi % 2
            e0 = k * _SG
            return [
                pltpu.async_copy(qew_hbm.at[c].at[region(vi)]
                                 .at[pl.ds(e0, _SG)], ews[p], semi[p]),
                pltpu.async_copy(qcol_hbm.at[c].at[region(vi)]
                                 .at[pl.ds(e0, _SG)], cols[p], semi[p]),
                pltpu.async_copy(qrow_hbm.at[c].at[region(vi)].at[k],
                                 rowi[vi % 4], semi[p]),
            ]

        def issue_gather(vi):
            p = vi % 2
            return [
                pltpu.async_copy(h_hbm.at[cols[p].at[pl.ds(j * _PM, _PM)]],
                                 rowsd[p].at[pl.ds(j * _PM, _PM)], semg[p])
                for j in range(_SK)
            ]

        def issue_scatter(vi):
            p = vi % 2
            return [
                pltpu.async_copy(rowsd[p].at[pl.ds(j * _PM, _PM)],
                                 agg_sh.at[rowi[vi % 4].at[j]], sems[p],
                                 add=True)
                for j in range(_SK)
            ]

        def scale(p):
            @pl.loop(0, _SG // _L)
            def _grp(g):
                g0 = g * _L
                ew16 = ews[p][pl.ds(g0, _L)]
                for k in range(_L):
                    w = _lane_bcast(ew16, k)
                    r = g0 + k
                    rowsd[p][r, pl.ds(0, _L)] = rowsd[p][r, pl.ds(0, _L)] * w
                    rowsd[p][r, pl.ds(_L, _L)] = (
                        rowsd[p][r, pl.ds(_L, _L)] * w)

        def wait_idx(vi):
            r, k = seq[vi]
            p = vi % 2
            e0 = k * _SG
            pltpu.make_async_copy(qew_hbm.at[c].at[region(vi)]
                                  .at[pl.ds(e0, _SG)], ews[p],
                                  semi[p]).wait()
            pltpu.make_async_copy(qcol_hbm.at[c].at[region(vi)]
                                  .at[pl.ds(e0, _SG)], cols[p],
                                  semi[p]).wait()
            pltpu.make_async_copy(qrow_hbm.at[c].at[region(vi)].at[k],
                                  rowi[vi % 4], semi[p]).wait()

        def wait_gather(vi):
            p = vi % 2
            for j in range(_SK):
                pltpu.make_async_copy(
                    h_hbm.at[cols[p].at[pl.ds(j * _PM, _PM)]],
                    rowsd[p].at[pl.ds(j * _PM, _PM)], semg[p]).wait()

        def wait_scatter(vi):
            p = vi % 2
            for j in range(_SK):
                pltpu.make_async_copy(
                    rowsd[p].at[pl.ds(j * _PM, _PM)],
                    agg_sh.at[rowi[vi % 4].at[j]], sems[p]).wait()

        def guarded(vi, fn):
            @pl.when(active(vi))
            def _():
                fn()

        def prolog():
            issue_idx(0)
        guarded(0, prolog)
        if nv > 1:
            def prolog1():
                issue_idx(1)
            guarded(1, prolog1)

        def prolog2():
            wait_idx(0)
            issue_gather(0)
        guarded(0, prolog2)

        for ci in range(nv):
            def w_gat(ci=ci):
                wait_gather(ci)
            guarded(ci, w_gat)
            if ci >= 1:
                def w_sca(ci=ci):
                    wait_scatter(ci - 1)
                guarded(ci - 1, w_sca)
            if ci + 1 < nv:
                def nxt(ci=ci):
                    wait_idx(ci + 1)
                    issue_gather(ci + 1)
                guarded(ci + 1, nxt)

            def body(ci=ci):
                scale(ci % 2)
                issue_scatter(ci)
            guarded(ci, body)
            if ci + 2 < nv:
                def pre(ci=ci):
                    issue_idx(ci + 2)
                guarded(ci + 2, pre)
        if nv >= 1:
            def tail():
                wait_scatter(nv - 1)
            guarded(nv - 1, tail)

        plsc.subcore_barrier()

        for hh in up_h:
            hh.wait()
        pltpu.sync_copy(agg_sh.at[pl.ds(s * own, own)], acc.at[pl.ds(0, own)])
        lanes = lax.iota(jnp.int32, _L)

        @pl.loop(0, own // _L)
        def _upd(g):
            ndx = lanes + jnp.full((_L,), g * _L, jnp.int32)
            inv16 = invb[pl.ds(g * _L, _L)]

            def jstep(j, outs):
                jf = jnp.full((_L,), j, jnp.int32)
                vj = plsc.load_gather(acc, [ndx, jf]) * inv16
                new = []
                for i in range(hdim):
                    w = wxb[j * hdim + i, pl.ds(0, _L)]
                    new.append(outs[i] + vj * w)
                return tuple(new)

            outs = lax.fori_loop(
                0, hdim, jstep,
                tuple(jnp.zeros((_L,), jnp.float32) for _ in range(hdim)))
            for i in range(hdim):
                ifl = jnp.full((_L,), i, jnp.int32)
                pre = outs[i] + plsc.load_gather(xpb, [ndx, ifl])
                t = jnp.exp(pre * 2.0)
                th = 1.0 - 2.0 / (t + 1.0)
                hv = plsc.load_gather(hb, [ndx, ifl])
                plsc.store_scatter(hb, [ndx, ifl],
                                   (1.0 - _LEAK) * hv + _LEAK * th)

        pltpu.sync_copy(hb, out_hbm.at[pl.ds(noff, own)])

    return kern


def _input_proj(x, w_in, b_in):
    t, n, f = x.shape
    hdim = w_in.shape[0]

    def body(x_ref, w_ref, b_ref, o_ref):
        xb = x_ref[0]
        o_ref[0] = (lax.dot_general(xb, w_ref[...], (((1,), (1,)), ((), ())),
                                    preferred_element_type=jnp.float32)
                    + b_ref[...])

    return pl.pallas_call(
        body,
        grid=(t,),
        in_specs=[
            pl.BlockSpec((1, n, f), lambda i: (i, 0, 0)),
            pl.BlockSpec((hdim, f), lambda i: (0, 0)),
            pl.BlockSpec((1, hdim), lambda i: (0, 0)),
        ],
        out_specs=pl.BlockSpec((1, n, hdim), lambda i: (i, 0, 0)),
        out_shape=jax.ShapeDtypeStruct((t, n, hdim), jnp.float32),
    )(x, w_in, b_in.reshape(1, hdim))


def _invdeg(degp):
    npad = degp.shape[1]

    def body(d_ref, o_ref):
        o_ref[...] = 1.0 / (d_ref[0:1] + d_ref[1:2] - 1.0)

    return pl.pallas_call(
        body,
        out_shape=jax.ShapeDtypeStruct((1, npad), jnp.float32),
    )(degp).reshape(npad)


def kernel(x, edge_index, edge_weight, W_in, b_in, W_h):
    t, n, _ = x.shape
    hdim = W_h.shape[0]
    e = edge_weight.shape[0]

    col = edge_index[0].astype(jnp.int32)
    row = edge_index[1].astype(jnp.int32)
    ew = edge_weight.astype(jnp.float32)

    block = _PG * _NW
    epad = ((e + block - 1) // block) * block
    if epad != e:
        fill = jnp.arange(epad - e, dtype=jnp.int32) % n
        col = jnp.concatenate([col, fill])
        row = jnp.concatenate([row, fill])
        ew = jnp.concatenate([ew, jnp.zeros((epad - e,), jnp.float32)])
    row2d = row.reshape(epad // _PM, _PM)
    ew2d = ew.reshape(epad // _PM, _PM)

    npad = ((n + 16 * _NS - 1) // (16 * _NS)) * (16 * _NS)
    ones = jnp.ones((npad,), jnp.float32)

    xproj = _input_proj(x, W_in, b_in)
    xproj = jnp.pad(xproj, ((0, 0), (0, npad - n), (0, 0)))
    degp = _degrees(row2d, ew2d, ones, npad)
    invd = _invdeg(degp)
    qcol, qrow, qew, qcnt = _bucket_edges(col, row2d, ew2d, npad)
    wxj = W_h.T.reshape(hdim * hdim, 1) * jnp.ones((1, _L), jnp.float32)
    zero_cnt = jnp.zeros((2, _NW, _L), jnp.int32)

    step_k = _make_step(npad, hdim)
    cnt_seq = jnp.concatenate(
        [zero_cnt[None], jnp.broadcast_to(qcnt, (t - 1,) + qcnt.shape)])
    h0 = jnp.zeros((npad, hdim), jnp.float32)

    def scan_body(h, xs):
        xp_t, cnts = xs
        h_new = step_k(qcol, qrow, qew, cnts, h, xp_t, invd, wxj)
        return h_new, h_new

    _, states = lax.scan(scan_body, h0, (xproj, cnt_seq))
    return states[:, :n, :]


# speedup vs baseline: 2.7347x; 2.7347x over previous
import dataclasses
import functools

import jax
import jax.numpy as jnp
from jax import lax
from jax.experimental import pallas as pl
from jax.experimental.pallas import tpu as pltpu
from jax.experimental.pallas import tpu_sc as plsc

_LEAK = 0.9

_NC = 2
_NS = 16
_NW = _NC * _NS
_L = 16

def _sc_params():
    cp = pltpu.CompilerParams()
    if "use_tc_tiling_on_sc" in pltpu.CompilerParams.__dataclass_fields__:
        cp = dataclasses.replace(cp, use_tc_tiling_on_sc=False)
    return cp


_M = 80
_K = 8
_G = _M * _K

_PM = 128
_PK = 8
_PG = _PM * _PK


def _lane_bcast(vec, lane):
    idx = jnp.full((_L,), lane, dtype=jnp.int32)
    dnums = lax.GatherDimensionNumbers(
        offset_dims=(), collapsed_slice_dims=(0,), start_index_map=(0,))
    return lax.gather(vec, idx[:, None], dnums, slice_sizes=(1,),
                      mode=lax.GatherScatterMode.PROMISE_IN_BOUNDS)


def _propagate(col, row2d, ew, h):
    n, hdim = h.shape
    epad = ew.shape[0]
    nb = epad // (_PG * _NW)
    stripe = n // _NS
    mesh = plsc.VectorSubcoreMesh(core_axis_name="c", subcore_axis_name="s")

    @functools.partial(
        pl.kernel,
        out_type=jax.ShapeDtypeStruct((_NC, n, hdim), jnp.float32),
        mesh=mesh,
        compiler_params=_sc_params(),
        scratch_types=(
            [pltpu.VMEM((_PG,), jnp.int32) for _ in range(3)]
            + [pltpu.VMEM((_PK, _PM), jnp.int32) for _ in range(4)]
            + [pltpu.VMEM((_PG,), jnp.float32) for _ in range(3)]
            + [pltpu.VMEM((_PG, hdim), jnp.float32) for _ in range(3)]
            + [pltpu.SemaphoreType.DMA for _ in range(9)]
            + [pltpu.VMEM_SHARED((n, hdim), jnp.float32)]
        ),
    )
    def kern(col_hbm, row_hbm, ew_hbm, h_hbm, out_hbm, *scr):
        cols = scr[0:3]
        rowi = scr[3:7]
        ews = scr[7:10]
        rowsd = scr[10:13]
        semi = scr[13:16]
        semg = scr[16:19]
        sems = scr[19:22]
        agg_sh = scr[22]
        c = lax.axis_index("c")
        s = lax.axis_index("s")
        wid = c * _NS + s
        hoff = pl.multiple_of(s * stripe, 8)
        pltpu.sync_copy(h_hbm.at[pl.ds(hoff, stripe)],
                        agg_sh.at[pl.ds(s * stripe, stripe)])
        plsc.subcore_barrier()

        def issue_idx(ci):
            b = wid + ci * _NW
            p = ci % 3
            e0 = pl.multiple_of(b * _PG, 8)
            r0 = pl.multiple_of(b * _PK, 8)
            return [
                pltpu.async_copy(ew_hbm.at[pl.ds(e0, _PG)], ews[p], semi[p]),
                pltpu.async_copy(col_hbm.at[pl.ds(e0, _PG)], cols[p], semi[p]),
                pltpu.async_copy(row_hbm.at[pl.ds(r0, _PK)], rowi[ci % 4],
                                 semi[p]),
            ]

        def issue_gather(ci):
            p = ci % 3
            return [
                pltpu.async_copy(h_hbm.at[cols[p].at[pl.ds(j * _PM, _PM)]],
                                 rowsd[p].at[pl.ds(j * _PM, _PM)], semg[p])
                for j in range(_PK)
            ]

        def issue_scatter(ci):
            p = ci % 3
            return [
                pltpu.async_copy(rowsd[p].at[pl.ds(j * _PM, _PM)],
                                 agg_sh.at[rowi[ci % 4].at[j]], sems[p],
                                 add=True)
                for j in range(_PK)
            ]

        def scale(p):
            @pl.loop(0, _PG // _L)
            def _grp(g):
                g0 = g * _L
                ew16 = ews[p][pl.ds(g0, _L)]
                for k in range(_L):
                    w = _lane_bcast(ew16, k)
                    r = g0 + k
                    rowsd[p][r, pl.ds(0, _L)] = rowsd[p][r, pl.ds(0, _L)] * w
                    rowsd[p][r, pl.ds(_L, _L)] = (
                        rowsd[p][r, pl.ds(_L, _L)] * w)

        def wait(handles):
            for hh in handles:
                hh.wait()

        idx_h = [None] * (nb + 2)
        gat_h = [None] * (nb + 1)
        sca_h = [None] * nb
        idx_h[0] = issue_idx(0)
        if nb > 1:
            idx_h[1] = issue_idx(1)
        wait(idx_h[0])
        gat_h[0] = issue_gather(0)
        for ci in range(nb):
            wait(gat_h[ci])
            if ci >= 2:
                wait(sca_h[ci - 2])
            if ci + 1 < nb:
                wait(idx_h[ci + 1])
                gat_h[ci + 1] = issue_gather(ci + 1)
            scale(ci % 3)
            sca_h[ci] = issue_scatter(ci)
            if ci + 2 < nb:
                idx_h[ci + 2] = issue_idx(ci + 2)
        if nb >= 2:
            wait(sca_h[nb - 2])
        wait(sca_h[nb - 1])

        plsc.subcore_barrier()
        pltpu.sync_copy(agg_sh.at[pl.ds(s * stripe, stripe)],
                        out_hbm.at[c].at[pl.ds(hoff, stripe)])

    return kern(col, row2d, ew, h)


def _degrees(row2d, ew2d, ones, npad):
    nrows = row2d.shape[0]
    nb = nrows // (_PK * _NW)
    dstripe = npad // _NS
    mesh = plsc.VectorSubcoreMesh(core_axis_name="c", subcore_axis_name="s")

    @functools.partial(
        pl.kernel,
        out_type=jax.ShapeDtypeStruct((_NC, npad), jnp.float32),
        mesh=mesh,
        compiler_params=_sc_params(),
        scratch_types=[
            pltpu.VMEM((nb, _PK, _PM), jnp.int32),
            pltpu.VMEM((nb, _PK, _PM), jnp.float32),
            pltpu.SemaphoreType.DMA,
            pltpu.SemaphoreType.DMA,
            pltpu.VMEM_SHARED((npad,), jnp.float32),
        ],
    )
    def kern(row_hbm, ew_hbm, ones_hbm, out_hbm, row_v, ew_v, semi, sems,
             deg_sh):
        c = lax.axis_index("c")
        s = lax.axis_index("s")
        wid = c * _NS + s
        doff = pl.multiple_of(s * dstripe, 8)
        loads = []
        for b in range(nb):
            r0 = pl.multiple_of((wid + b * _NW) * _PK, 8)
            loads.append(pltpu.async_copy(row_hbm.at[pl.ds(r0, _PK)],
                                          row_v.at[b], semi))
            loads.append(pltpu.async_copy(ew_hbm.at[pl.ds(r0, _PK)],
                                          ew_v.at[b], semi))
        pltpu.sync_copy(ones_hbm.at[pl.ds(doff, dstripe)],
                        deg_sh.at[pl.ds(s * dstripe, dstripe)])
        plsc.subcore_barrier()
        for hh in loads:
            hh.wait()
        scats = []
        for b in range(nb):
            for j in range(_PK):
                scats.append(pltpu.async_copy(
                    ew_v.at[b].at[j], deg_sh.at[row_v.at[b].at[j]], sems,
                    add=True))
        for hh in scats:
            hh.wait()

        plsc.subcore_barrier()
        pltpu.sync_copy(deg_sh.at[pl.ds(s * dstripe, dstripe)],
                        out_hbm.at[c].at[pl.ds(doff, dstripe)])

    return kern(row2d, ew2d, ones)


def _input_proj(x, w_in, b_in):
    t, n, f = x.shape
    hdim = w_in.shape[0]

    def body(x_ref, w_ref, b_ref, o_ref):
        xb = x_ref[0]
        o_ref[0] = (lax.dot_general(xb, w_ref[...], (((1,), (1,)), ((), ())),
                                    preferred_element_type=jnp.float32)
                    + b_ref[...])

    return pl.pallas_call(
        body,
        grid=(t,),
        in_specs=[
            pl.BlockSpec((1, n, f), lambda i: (i, 0, 0)),
            pl.BlockSpec((hdim, f), lambda i: (0, 0)),
            pl.BlockSpec((1, hdim), lambda i: (0, 0)),
        ],
        out_specs=pl.BlockSpec((1, n, hdim), lambda i: (i, 0, 0)),
        out_shape=jax.ShapeDtypeStruct((t, n, hdim), jnp.float32),
    )(x, w_in, b_in.reshape(1, hdim))


def _update(xproj_t, h, aggp, degp, w_h):
    n, hdim = h.shape

    def body(xp_ref, h_ref, ag_ref, dg_ref, w_ref, o_ref):
        hcur = h_ref[...]
        raw = ag_ref[0] + ag_ref[1] - hcur
        deg = dg_ref[0] + dg_ref[1] - 1.0
        tot = raw * (1.0 / deg)[:, None]
        pre = xp_ref[...] + lax.dot_general(
            tot, w_ref[...], (((1,), (1,)), ((), ())),
            preferred_element_type=jnp.float32)
        o_ref[...] = (1.0 - _LEAK) * hcur + _LEAK * jnp.tanh(pre)

    return pl.pallas_call(
        body,
        out_shape=jax.ShapeDtypeStruct((n, hdim), jnp.float32),
    )(xproj_t, h, aggp, degp, w_h)


def kernel(x, edge_index, edge_weight, W_in, b_in, W_h):
    t, n, _ = x.shape
    hdim = W_h.shape[0]
    e = edge_weight.shape[0]

    col = edge_index[0].astype(jnp.int32)
    row = edge_index[1].astype(jnp.int32)
    ew = edge_weight.astype(jnp.float32)

    block = _PG * _NW
    epad = ((e + block - 1) // block) * block
    if epad != e:
        fill = jnp.arange(epad - e, dtype=jnp.int32) % n
        col = jnp.concatenate([col, fill])
        row = jnp.concatenate([row, fill])
        ew = jnp.concatenate([ew, jnp.zeros((epad - e,), jnp.float32)])
    row2d = row.reshape(epad // _PM, _PM)
    ew2d = ew.reshape(epad // _PM, _PM)

    npad = ((n + 8 * _NS - 1) // (8 * _NS)) * (8 * _NS)
    ones = jnp.ones((npad,), jnp.float32)

    xproj = _input_proj(x, W_in, b_in)
    xproj = jnp.pad(xproj, ((0, 0), (0, npad - n), (0, 0)))
    degp = _degrees(row2d, ew2d, ones, npad)

    zeros_h = jnp.zeros((npad, hdim), jnp.float32)
    zeros_agg = jnp.zeros((_NC, npad, hdim), jnp.float32)
    zeros_deg = jnp.zeros((_NC, npad), jnp.float32)

    h = _update(xproj[0], zeros_h, zeros_agg, zeros_deg, W_h)
    outs = [h]
    for step in range(1, t):
        aggp = _propagate(col, row2d, ew, h)
        h = _update(xproj[step], h, aggp, degp, W_h)
        outs.append(h)
    return jnp.stack(outs)[:, :n, :]
